# baseline (speedup 1.0000x reference)
import functools

import jax
import jax.numpy as jnp
from jax import lax
from jax.experimental import pallas as pl
from jax.experimental.pallas import tpu as pltpu
from jax.experimental.pallas import tpu_sc as plsc

N, E, D, H, DH, D_EDGE = 10000, 320000, 128, 8, 16, 4
ALPHA = 0.2
EPS = 1e-5

NC, NS, L = 2, 16, 16
NW = NC * NS
CHUNK = 80
GROUPS = CHUNK // L
EDGES_PER_TILE = E // NW
CHUNKS_PER_TILE = EDGES_PER_TILE // CHUNK
IDXBLK = 25
NBLK = CHUNKS_PER_TILE // IDXBLK
N_PAD = 10240
ROWS_PER_TILE = N_PAD // NS
STRIDE = 136
PRODW = (CHUNK - 1) * STRIDE + D + 8


def _qkv_body(x_ref, wq_ref, wk_ref, wv_ref, q_ref, k_ref, v_ref):
    xb = x_ref[...]
    q_ref[...] = jnp.dot(xb, wq_ref[...], preferred_element_type=jnp.float32)
    k_ref[...] = jnp.dot(xb, wk_ref[...], preferred_element_type=jnp.float32)
    v_ref[...] = jnp.dot(xb, wv_ref[...], preferred_element_type=jnp.float32)


def _qkv_call(x, W_q, W_k, W_v):
    blk = 1000
    grid = (N // blk,)
    return pl.pallas_call(
        _qkv_body,
        grid=grid,
        in_specs=[
            pl.BlockSpec((blk, D), lambda i: (i, 0)),
            pl.BlockSpec((D, D), lambda i: (0, 0)),
            pl.BlockSpec((D, D), lambda i: (0, 0)),
            pl.BlockSpec((D, D), lambda i: (0, 0)),
        ],
        out_specs=[
            pl.BlockSpec((blk, D), lambda i: (i, 0)),
            pl.BlockSpec((blk, D), lambda i: (i, 0)),
            pl.BlockSpec((blk, D), lambda i: (i, 0)),
        ],
        out_shape=[
            jax.ShapeDtypeStruct((N, D), jnp.float32),
            jax.ShapeDtypeStruct((N, D), jnp.float32),
            jax.ShapeDtypeStruct((N, D), jnp.float32),
        ],
    )(x, W_q, W_k, W_v)


def _ew_body(a0, a1, a2, a3, o_ref):
    ssum = a0[...] + a1[...] + a2[...] + a3[...]
    o_ref[...] = 1.0 / (1.0 + jnp.exp(-ssum))


def _ew_call(edge_attr):
    rows = E // D
    parts = [edge_attr[:, j].reshape(rows, D) for j in range(D_EDGE)]
    return pl.pallas_call(
        _ew_body,
        grid=(1,),
        in_specs=[pl.BlockSpec((rows, D), lambda i: (0, 0))] * D_EDGE,
        out_specs=pl.BlockSpec((rows, D), lambda i: (0, 0)),
        out_shape=jax.ShapeDtypeStruct((rows, D), jnp.float32),
    )(*parts)


def _sc_edge_body(q_hbm, k_hbm, v_hbm, row_hbm, col_hbm, ew_hbm, z_hbm,
                  out_hbm, rowv, colv, ew_v, q_v, kva, kvb, prod, ps, acc,
                  sem_q, sem_k, sem_v, sem_s):
    c = lax.axis_index("c")
    s = lax.axis_index("s")
    tid = c * NS + s

    pltpu.sync_copy(z_hbm.at[pl.ds(s * ROWS_PER_TILE, ROWS_PER_TILE)],
                    acc.at[pl.ds(s * ROWS_PER_TILE, ROWS_PER_TILE)])
    plsc.subcore_barrier()

    iota = lax.iota(jnp.int32, L)
    ib = iota * STRIDE

    def drain_scatter():
        pltpu.make_async_copy(kva, acc.at[colv.at[0]], sem_s).wait()

    def process_chunk(j, kv_this, kv_other):
        pltpu.make_async_copy(q_hbm.at[pl.ds(0, CHUNK)], q_v, sem_q).wait()
        pltpu.make_async_copy(k_hbm.at[pl.ds(0, CHUNK)], kv_this,
                              sem_k).wait()

        def mul_body(e):
            base = e * STRIDE
            for h8 in range(H):
                qrow = q_v[e, pl.ds(h8 * DH, DH)]
                krow = kv_this[e, pl.ds(h8 * DH, DH)]
                prod[pl.ds(base + h8 * DH, DH)] = qrow * krow

        plsc.parallel_loop(0, CHUNK // 2, unroll=2)(mul_body)
        pltpu.async_copy(v_hbm.at[rowv.at[j, pl.ds(0, CHUNK // 2)]],
                         kv_this.at[pl.ds(0, CHUNK // 2)], sem_v)
        plsc.parallel_loop(CHUNK // 2, CHUNK, unroll=2)(mul_body)
        pltpu.async_copy(v_hbm.at[rowv.at[j, pl.ds(CHUNK // 2, CHUNK // 2)]],
                         kv_this.at[pl.ds(CHUNK // 2, CHUNK // 2)], sem_v)

        @pl.when(j < IDXBLK - 1)
        def _():
            pltpu.async_copy(q_hbm.at[rowv.at[j + 1]], q_v, sem_q)

        def score_body(g):
            gbase = g * (L * STRIDE)
            ewv = ew_v[j, pl.ds(g * L, L)]
            scores = []
            for h in range(H):
                parts = [jnp.zeros((L,), jnp.float32) for _ in range(4)]
                for d in range(DH):
                    idx = ib + jnp.full((L,), gbase + h * DH + d, jnp.int32)
                    parts[d % 4] = parts[d % 4] + plsc.load_gather(
                        prod, [idx])
                acc_s = (parts[0] + parts[1]) + (parts[2] + parts[3])
                sc = acc_s * 0.25
                sc = jnp.maximum(sc, ALPHA * sc)
                scores.append(sc * ewv)
            m = jnp.maximum(
                jnp.maximum(jnp.maximum(scores[0], scores[1]),
                            jnp.maximum(scores[2], scores[3])),
                jnp.maximum(jnp.maximum(scores[4], scores[5]),
                            jnp.maximum(scores[6], scores[7])))
            es = [jnp.exp(sc - m) for sc in scores]
            ssum = ((es[0] + es[1]) + (es[2] + es[3])) + \
                   ((es[4] + es[5]) + (es[6] + es[7]))
            rinv = 1.0 / ssum
            for h in range(H):
                ps[g * H + h, :] = es[h] * rinv
            return 0

        lax.fori_loop(0, GROUPS, lambda g, cy: score_body(g) or cy, 0)

        pltpu.make_async_copy(v_hbm.at[pl.ds(0, CHUNK)], kv_this,
                              sem_v).wait()

        @pl.when(j > 0)
        def _():
            drain_scatter()

        @pl.when(j < IDXBLK - 1)
        def _():
            pltpu.async_copy(k_hbm.at[colv.at[j + 1]], kv_other, sem_k)

        def scale_body(g):
            probs = [ps[g * H + h, :] for h in range(H)]

            dnums = lax.GatherDimensionNumbers(
                offset_dims=(), collapsed_slice_dims=(0,),
                start_index_map=(0,))
            for lane in range(L):
                e_row = g * L + lane
                bidx = jnp.full((L, 1), lane, jnp.int32)
                for h in range(H):
                    pb = lax.gather(
                        probs[h], bidx, dnums, (1,),
                        mode=lax.GatherScatterMode.PROMISE_IN_BOUNDS)
                    vrow = kv_this[e_row, pl.ds(h * DH, DH)]
                    kv_this[e_row, pl.ds(h * DH, DH)] = vrow * pb
            return 0

        lax.fori_loop(0, GROUPS, lambda g, cy: scale_body(g) or cy, 0)

        pltpu.async_copy(kv_this, acc.at[colv.at[j]], sem_s, add=True)

    def blk_body(b, carry0):
        @pl.when(b > 0)
        def _():
            drain_scatter()

        gblk = tid * NBLK + b
        pltpu.sync_copy(row_hbm.at[gblk], rowv)
        pltpu.sync_copy(col_hbm.at[gblk], colv)
        pltpu.sync_copy(ew_hbm.at[gblk], ew_v)
        pltpu.async_copy(q_hbm.at[rowv.at[0]], q_v, sem_q)
        pltpu.async_copy(k_hbm.at[colv.at[0]], kva, sem_k)

        def pair_body(j2, carry):
            process_chunk(2 * j2, kva, kvb)
            process_chunk(2 * j2 + 1, kvb, kva)
            return carry

        lax.fori_loop(0, IDXBLK // 2, pair_body, 0)
        process_chunk(IDXBLK - 1, kva, kvb)
        return carry0

    lax.fori_loop(0, NBLK, blk_body, 0)
    drain_scatter()

    plsc.subcore_barrier()
    pltpu.sync_copy(acc.at[pl.ds(s * ROWS_PER_TILE, ROWS_PER_TILE)],
                    out_hbm.at[c, pl.ds(s * ROWS_PER_TILE, ROWS_PER_TILE)])


def _sc_edge_call(q_tab, k_tab, v_tab, row3d, col3d, ew3d, zeros_n):
    mesh = plsc.VectorSubcoreMesh(core_axis_name="c", subcore_axis_name="s")
    fn = functools.partial(
        pl.kernel,
        mesh=mesh,
        compiler_params=pltpu.CompilerParams(use_tc_tiling_on_sc=False,
                                             needs_layout_passes=False),
        out_type=jax.ShapeDtypeStruct((NC, N_PAD, D), jnp.float32),
        scratch_types=[
            pltpu.VMEM((IDXBLK, CHUNK), jnp.int32),
            pltpu.VMEM((IDXBLK, CHUNK), jnp.int32),
            pltpu.VMEM((IDXBLK, CHUNK), jnp.float32),
            pltpu.VMEM((CHUNK, D), jnp.float32),
            pltpu.VMEM((CHUNK, D), jnp.float32),
            pltpu.VMEM((CHUNK, D), jnp.float32),
            pltpu.VMEM((PRODW,), jnp.float32),
            pltpu.VMEM((GROUPS * H, L), jnp.float32),
            pltpu.VMEM_SHARED((N_PAD, D), jnp.float32),
            pltpu.SemaphoreType.DMA,
            pltpu.SemaphoreType.DMA,
            pltpu.SemaphoreType.DMA,
            pltpu.SemaphoreType.DMA,
        ],
    )(_sc_edge_body)
    return fn(q_tab, k_tab, v_tab, row3d, col3d, ew3d, zeros_n)


def _out_body(p0_ref, p1_ref, x_ref, wo_ref, bo_ref, g_ref, b_ref, o_ref):
    pb = p0_ref[0] + p1_ref[0]
    y = jnp.dot(pb, wo_ref[...], preferred_element_type=jnp.float32)
    y = y + bo_ref[...] + x_ref[...]
    mu = jnp.mean(y, axis=-1, keepdims=True)
    yc = y - mu
    var = jnp.mean(yc * yc, axis=-1, keepdims=True)
    o_ref[...] = yc * lax.rsqrt(var + EPS) * g_ref[...] + b_ref[...]


def _out_call(partials, x, W_o, b_o, ln_g, ln_b):
    blk = 1000
    grid = (N // blk,)
    return pl.pallas_call(
        _out_body,
        grid=grid,
        in_specs=[
            pl.BlockSpec((1, blk, D), lambda i: (0, i, 0)),
            pl.BlockSpec((1, blk, D), lambda i: (1, i, 0)),
            pl.BlockSpec((blk, D), lambda i: (i, 0)),
            pl.BlockSpec((D, D), lambda i: (0, 0)),
            pl.BlockSpec((1, D), lambda i: (0, 0)),
            pl.BlockSpec((1, D), lambda i: (0, 0)),
            pl.BlockSpec((1, D), lambda i: (0, 0)),
        ],
        out_specs=pl.BlockSpec((blk, D), lambda i: (i, 0)),
        out_shape=jax.ShapeDtypeStruct((N, D), jnp.float32),
    )(partials, partials, x, W_o, b_o, ln_g, ln_b)


def kernel(x, edge_index, edge_attr, W_q, W_k, W_v, W_o, b_o, ln_g, ln_b):
    row3d = edge_index[0].reshape(NW * NBLK, IDXBLK, CHUNK)
    col3d = edge_index[1].reshape(NW * NBLK, IDXBLK, CHUNK)
    zeros_n = jnp.zeros((N_PAD, D), jnp.float32)
    ew3d = _ew_call(edge_attr).reshape(NW * NBLK, IDXBLK, CHUNK)
    q_tab, k_tab, v_tab = _qkv_call(x, W_q, W_k, W_v)
    partials = _sc_edge_call(q_tab, k_tab, v_tab, row3d, col3d, ew3d, zeros_n)
    return _out_call(partials, x, W_o,
                     b_o.reshape(1, D), ln_g.reshape(1, D), ln_b.reshape(1, D))


# speedup vs baseline: 1.0200x; 1.0200x over previous
import functools

import jax
import jax.numpy as jnp
from jax import lax
from jax.experimental import pallas as pl
from jax.experimental.pallas import tpu as pltpu
from jax.experimental.pallas import tpu_sc as plsc

N, E, D, H, DH, D_EDGE = 10000, 320000, 128, 8, 16, 4
ALPHA = 0.2
EPS = 1e-5

NC, NS, L = 2, 16, 16
NW = NC * NS
CHUNK = 80
GROUPS = CHUNK // L
EDGES_PER_TILE = E // NW
CHUNKS_PER_TILE = EDGES_PER_TILE // CHUNK
IDXBLK = 25
NBLK = CHUNKS_PER_TILE // IDXBLK
N_PAD = 10240
ROWS_PER_TILE = N_PAD // NS
STRIDE = 136
PRODW = (CHUNK - 1) * STRIDE + D + 8


def _qkv_body(x_ref, wq_ref, wk_ref, wv_ref, q_ref, k_ref, v_ref):
    xb = x_ref[...]
    q_ref[...] = jnp.dot(xb, wq_ref[...], preferred_element_type=jnp.float32)
    k_ref[...] = jnp.dot(xb, wk_ref[...], preferred_element_type=jnp.float32)
    v_ref[...] = jnp.dot(xb, wv_ref[...], preferred_element_type=jnp.float32)


def _qkv_call(x, W_q, W_k, W_v):
    blk = 1000
    grid = (N // blk,)
    return pl.pallas_call(
        _qkv_body,
        grid=grid,
        in_specs=[
            pl.BlockSpec((blk, D), lambda i: (i, 0)),
            pl.BlockSpec((D, D), lambda i: (0, 0)),
            pl.BlockSpec((D, D), lambda i: (0, 0)),
            pl.BlockSpec((D, D), lambda i: (0, 0)),
        ],
        out_specs=[
            pl.BlockSpec((blk, D), lambda i: (i, 0)),
            pl.BlockSpec((blk, D), lambda i: (i, 0)),
            pl.BlockSpec((blk, D), lambda i: (i, 0)),
        ],
        out_shape=[
            jax.ShapeDtypeStruct((N, D), jnp.float32),
            jax.ShapeDtypeStruct((N, D), jnp.float32),
            jax.ShapeDtypeStruct((N, D), jnp.float32),
        ],
    )(x, W_q, W_k, W_v)


def _ew_body(a0, a1, a2, a3, o_ref):
    ssum = a0[...] + a1[...] + a2[...] + a3[...]
    o_ref[...] = 1.0 / (1.0 + jnp.exp(-ssum))


def _ew_call(edge_attr):
    rows = E // D
    parts = [edge_attr[:, j].reshape(rows, D) for j in range(D_EDGE)]
    return pl.pallas_call(
        _ew_body,
        grid=(1,),
        in_specs=[pl.BlockSpec((rows, D), lambda i: (0, 0))] * D_EDGE,
        out_specs=pl.BlockSpec((rows, D), lambda i: (0, 0)),
        out_shape=jax.ShapeDtypeStruct((rows, D), jnp.float32),
    )(*parts)


def _sc_edge_body(q_hbm, k_hbm, v_hbm, row_hbm, col_hbm, ew_hbm, z_hbm,
                  out_hbm, rowv, colv, ew_v, q_v, kva, kvb, prod, ps, acc,
                  sem_q, sem_k, sem_v, sem_s):
    c = lax.axis_index("c")
    s = lax.axis_index("s")
    tid = c * NS + s

    pltpu.sync_copy(z_hbm.at[pl.ds(s * ROWS_PER_TILE, ROWS_PER_TILE)],
                    acc.at[pl.ds(s * ROWS_PER_TILE, ROWS_PER_TILE)])
    plsc.subcore_barrier()

    iota = lax.iota(jnp.int32, L)
    ib = iota * STRIDE

    def drain_scatter():
        pltpu.make_async_copy(kva, acc.at[colv.at[0]], sem_s).wait()

    def process_chunk(j, kv_this, kv_other):
        pltpu.make_async_copy(q_hbm.at[pl.ds(0, CHUNK)], q_v, sem_q).wait()
        pltpu.make_async_copy(k_hbm.at[pl.ds(0, CHUNK)], kv_this,
                              sem_k).wait()

        def mul_body(e):
            base = e * STRIDE
            for h8 in range(H):
                qrow = q_v[e, pl.ds(h8 * DH, DH)]
                krow = kv_this[e, pl.ds(h8 * DH, DH)]
                prod[pl.ds(base + h8 * DH, DH)] = qrow * krow

        plsc.parallel_loop(0, CHUNK // 2, unroll=2)(mul_body)
        pltpu.async_copy(v_hbm.at[rowv.at[j, pl.ds(0, CHUNK // 2)]],
                         kv_this.at[pl.ds(0, CHUNK // 2)], sem_v)
        plsc.parallel_loop(CHUNK // 2, CHUNK, unroll=2)(mul_body)
        pltpu.async_copy(v_hbm.at[rowv.at[j, pl.ds(CHUNK // 2, CHUNK // 2)]],
                         kv_this.at[pl.ds(CHUNK // 2, CHUNK // 2)], sem_v)

        @pl.when(j < IDXBLK - 1)
        def _():
            pltpu.async_copy(q_hbm.at[rowv.at[j + 1]], q_v, sem_q)

        def score_body(g):
            gbase = g * (L * STRIDE)
            ewv = ew_v[j, pl.ds(g * L, L)]
            for h in range(H):
                parts = [jnp.zeros((L,), jnp.float32) for _ in range(4)]
                for d in range(DH):
                    idx = ib + jnp.full((L,), gbase + h * DH + d, jnp.int32)
                    parts[d % 4] = parts[d % 4] + plsc.load_gather(
                        prod, [idx])
                acc_s = (parts[0] + parts[1]) + (parts[2] + parts[3])
                sc = acc_s * 0.25
                sc = jnp.maximum(sc, ALPHA * sc)
                sc = sc * ewv
                ps[g * H + h, :] = sc
            return 0

        lax.fori_loop(0, GROUPS, lambda g, cy: score_body(g) or cy, 0)

        pltpu.make_async_copy(v_hbm.at[pl.ds(0, CHUNK)], kv_this,
                              sem_v).wait()

        @pl.when(j > 0)
        def _():
            drain_scatter()

        @pl.when(j < IDXBLK - 1)
        def _():
            pltpu.async_copy(k_hbm.at[colv.at[j + 1]], kv_other, sem_k)

        def scale_body(g):
            scores = [ps[g * H + h, :] for h in range(H)]
            m = jnp.maximum(
                jnp.maximum(jnp.maximum(scores[0], scores[1]),
                            jnp.maximum(scores[2], scores[3])),
                jnp.maximum(jnp.maximum(scores[4], scores[5]),
                            jnp.maximum(scores[6], scores[7])))
            es = [jnp.exp(sc - m) for sc in scores]
            ssum = ((es[0] + es[1]) + (es[2] + es[3])) + \
                   ((es[4] + es[5]) + (es[6] + es[7]))
            rinv = 1.0 / ssum
            probs = [e_h * rinv for e_h in es]

            dnums = lax.GatherDimensionNumbers(
                offset_dims=(), collapsed_slice_dims=(0,),
                start_index_map=(0,))
            for lane in range(L):
                e_row = g * L + lane
                bidx = jnp.full((L, 1), lane, jnp.int32)
                for h in range(H):
                    pb = lax.gather(
                        probs[h], bidx, dnums, (1,),
                        mode=lax.GatherScatterMode.PROMISE_IN_BOUNDS)
                    vrow = kv_this[e_row, pl.ds(h * DH, DH)]
                    kv_this[e_row, pl.ds(h * DH, DH)] = vrow * pb
            return 0

        lax.fori_loop(0, GROUPS, lambda g, cy: scale_body(g) or cy, 0)

        pltpu.async_copy(kv_this, acc.at[colv.at[j]], sem_s, add=True)

    def blk_body(b, carry0):
        @pl.when(b > 0)
        def _():
            drain_scatter()

        gblk = tid * NBLK + b
        pltpu.sync_copy(row_hbm.at[gblk], rowv)
        pltpu.sync_copy(col_hbm.at[gblk], colv)
        pltpu.sync_copy(ew_hbm.at[gblk], ew_v)
        pltpu.async_copy(q_hbm.at[rowv.at[0]], q_v, sem_q)
        pltpu.async_copy(k_hbm.at[colv.at[0]], kva, sem_k)

        def pair_body(j2, carry):
            process_chunk(2 * j2, kva, kvb)
            process_chunk(2 * j2 + 1, kvb, kva)
            return carry

        lax.fori_loop(0, IDXBLK // 2, pair_body, 0)
        process_chunk(IDXBLK - 1, kva, kvb)
        return carry0

    lax.fori_loop(0, NBLK, blk_body, 0)
    drain_scatter()

    plsc.subcore_barrier()
    pltpu.sync_copy(acc.at[pl.ds(s * ROWS_PER_TILE, ROWS_PER_TILE)],
                    out_hbm.at[c, pl.ds(s * ROWS_PER_TILE, ROWS_PER_TILE)])


def _sc_edge_call(q_tab, k_tab, v_tab, row3d, col3d, ew3d, zeros_n):
    mesh = plsc.VectorSubcoreMesh(core_axis_name="c", subcore_axis_name="s")
    fn = functools.partial(
        pl.kernel,
        mesh=mesh,
        compiler_params=pltpu.CompilerParams(use_tc_tiling_on_sc=False,
                                             needs_layout_passes=False),
        out_type=jax.ShapeDtypeStruct((NC, N_PAD, D), jnp.float32),
        scratch_types=[
            pltpu.VMEM((IDXBLK, CHUNK), jnp.int32),
            pltpu.VMEM((IDXBLK, CHUNK), jnp.int32),
            pltpu.VMEM((IDXBLK, CHUNK), jnp.float32),
            pltpu.VMEM((CHUNK, D), jnp.float32),
            pltpu.VMEM((CHUNK, D), jnp.float32),
            pltpu.VMEM((CHUNK, D), jnp.float32),
            pltpu.VMEM((PRODW,), jnp.float32),
            pltpu.VMEM((GROUPS * H, L), jnp.float32),
            pltpu.VMEM_SHARED((N_PAD, D), jnp.float32),
            pltpu.SemaphoreType.DMA,
            pltpu.SemaphoreType.DMA,
            pltpu.SemaphoreType.DMA,
            pltpu.SemaphoreType.DMA,
        ],
    )(_sc_edge_body)
    return fn(q_tab, k_tab, v_tab, row3d, col3d, ew3d, zeros_n)


def _out_body(p0_ref, p1_ref, x_ref, wo_ref, bo_ref, g_ref, b_ref, o_ref):
    pb = p0_ref[0] + p1_ref[0]
    y = jnp.dot(pb, wo_ref[...], preferred_element_type=jnp.float32)
    y = y + bo_ref[...] + x_ref[...]
    mu = jnp.mean(y, axis=-1, keepdims=True)
    yc = y - mu
    var = jnp.mean(yc * yc, axis=-1, keepdims=True)
    o_ref[...] = yc * lax.rsqrt(var + EPS) * g_ref[...] + b_ref[...]


def _out_call(partials, x, W_o, b_o, ln_g, ln_b):
    blk = 1000
    grid = (N // blk,)
    return pl.pallas_call(
        _out_body,
        grid=grid,
        in_specs=[
            pl.BlockSpec((1, blk, D), lambda i: (0, i, 0)),
            pl.BlockSpec((1, blk, D), lambda i: (1, i, 0)),
            pl.BlockSpec((blk, D), lambda i: (i, 0)),
            pl.BlockSpec((D, D), lambda i: (0, 0)),
            pl.BlockSpec((1, D), lambda i: (0, 0)),
            pl.BlockSpec((1, D), lambda i: (0, 0)),
            pl.BlockSpec((1, D), lambda i: (0, 0)),
        ],
        out_specs=pl.BlockSpec((blk, D), lambda i: (i, 0)),
        out_shape=jax.ShapeDtypeStruct((N, D), jnp.float32),
    )(partials, partials, x, W_o, b_o, ln_g, ln_b)


def kernel(x, edge_index, edge_attr, W_q, W_k, W_v, W_o, b_o, ln_g, ln_b):
    row3d = edge_index[0].reshape(NW * NBLK, IDXBLK, CHUNK)
    col3d = edge_index[1].reshape(NW * NBLK, IDXBLK, CHUNK)
    zeros_n = jnp.zeros((N_PAD, D), jnp.float32)
    ew3d = _ew_call(edge_attr).reshape(NW * NBLK, IDXBLK, CHUNK)
    q_tab, k_tab, v_tab = _qkv_call(x, W_q, W_k, W_v)
    partials = _sc_edge_call(q_tab, k_tab, v_tab, row3d, col3d, ew3d, zeros_n)
    return _out_call(partials, x, W_o,
                     b_o.reshape(1, D), ln_g.reshape(1, D), ln_b.reshape(1, D))


# speedup vs baseline: 1.0275x; 1.0073x over previous
import functools

import jax
import jax.numpy as jnp
from jax import lax
from jax.experimental import pallas as pl
from jax.experimental.pallas import tpu as pltpu
from jax.experimental.pallas import tpu_sc as plsc

N, E, D, H, DH, D_EDGE = 10000, 320000, 128, 8, 16, 4
ALPHA = 0.2
EPS = 1e-5

NC, NS, L = 2, 16, 16
NW = NC * NS
CHUNK = 80
GROUPS = CHUNK // L
EDGES_PER_TILE = E // NW
CHUNKS_PER_TILE = EDGES_PER_TILE // CHUNK
IDXBLK = 25
NBLK = CHUNKS_PER_TILE // IDXBLK
N_PAD = 10240
ROWS_PER_TILE = N_PAD // NS
STRIDE = 137
PRODW = (CHUNK - 1) * STRIDE + D + 8


def _qkv_body(x_ref, wq_ref, wk_ref, wv_ref, q_ref, k_ref, v_ref):
    xb = x_ref[...]
    q_ref[...] = jnp.dot(xb, wq_ref[...], preferred_element_type=jnp.float32)
    k_ref[...] = jnp.dot(xb, wk_ref[...], preferred_element_type=jnp.float32)
    v_ref[...] = jnp.dot(xb, wv_ref[...], preferred_element_type=jnp.float32)


def _qkv_call(x, W_q, W_k, W_v):
    blk = 1000
    grid = (N // blk,)
    return pl.pallas_call(
        _qkv_body,
        grid=grid,
        in_specs=[
            pl.BlockSpec((blk, D), lambda i: (i, 0)),
            pl.BlockSpec((D, D), lambda i: (0, 0)),
            pl.BlockSpec((D, D), lambda i: (0, 0)),
            pl.BlockSpec((D, D), lambda i: (0, 0)),
        ],
        out_specs=[
            pl.BlockSpec((blk, D), lambda i: (i, 0)),
            pl.BlockSpec((blk, D), lambda i: (i, 0)),
            pl.BlockSpec((blk, D), lambda i: (i, 0)),
        ],
        out_shape=[
            jax.ShapeDtypeStruct((N, D), jnp.float32),
            jax.ShapeDtypeStruct((N, D), jnp.float32),
            jax.ShapeDtypeStruct((N, D), jnp.float32),
        ],
    )(x, W_q, W_k, W_v)


def _ew_body(a0, a1, a2, a3, o_ref):
    ssum = a0[...] + a1[...] + a2[...] + a3[...]
    o_ref[...] = 1.0 / (1.0 + jnp.exp(-ssum))


def _ew_call(edge_attr):
    rows = E // D
    parts = [edge_attr[:, j].reshape(rows, D) for j in range(D_EDGE)]
    return pl.pallas_call(
        _ew_body,
        grid=(1,),
        in_specs=[pl.BlockSpec((rows, D), lambda i: (0, 0))] * D_EDGE,
        out_specs=pl.BlockSpec((rows, D), lambda i: (0, 0)),
        out_shape=jax.ShapeDtypeStruct((rows, D), jnp.float32),
    )(*parts)


def _sc_edge_body(q_hbm, k_hbm, v_hbm, row_hbm, col_hbm, ew_hbm, z_hbm,
                  out_hbm, rowv, colv, ew_v, q_v, kva, kvb, prod, ps, acc,
                  sem_q, sem_k, sem_v, sem_s):
    c = lax.axis_index("c")
    s = lax.axis_index("s")
    tid = c * NS + s

    pltpu.sync_copy(z_hbm.at[pl.ds(s * ROWS_PER_TILE, ROWS_PER_TILE)],
                    acc.at[pl.ds(s * ROWS_PER_TILE, ROWS_PER_TILE)])
    plsc.subcore_barrier()

    iota = lax.iota(jnp.int32, L)
    ib = iota * STRIDE

    def drain_scatter():
        pltpu.make_async_copy(kva, acc.at[colv.at[0]], sem_s).wait()

    def process_chunk(j, kv_this, kv_other):
        pltpu.make_async_copy(q_hbm.at[pl.ds(0, CHUNK)], q_v, sem_q).wait()
        pltpu.make_async_copy(k_hbm.at[pl.ds(0, CHUNK)], kv_this,
                              sem_k).wait()

        def mul_body(e):
            base = e * STRIDE
            for h8 in range(H):
                qrow = q_v[e, pl.ds(h8 * DH, DH)]
                krow = kv_this[e, pl.ds(h8 * DH, DH)]
                prod[pl.ds(base + h8 * DH, DH)] = qrow * krow

        plsc.parallel_loop(0, CHUNK // 2, unroll=2)(mul_body)
        pltpu.async_copy(v_hbm.at[rowv.at[j, pl.ds(0, CHUNK // 2)]],
                         kv_this.at[pl.ds(0, CHUNK // 2)], sem_v)
        plsc.parallel_loop(CHUNK // 2, CHUNK, unroll=2)(mul_body)
        pltpu.async_copy(v_hbm.at[rowv.at[j, pl.ds(CHUNK // 2, CHUNK // 2)]],
                         kv_this.at[pl.ds(CHUNK // 2, CHUNK // 2)], sem_v)

        @pl.when(j < IDXBLK - 1)
        def _():
            pltpu.async_copy(q_hbm.at[rowv.at[j + 1]], q_v, sem_q)

        def score_body(g):
            gbase = g * (L * STRIDE)
            ewv = ew_v[j, pl.ds(g * L, L)]
            for h in range(H):
                parts = [jnp.zeros((L,), jnp.float32) for _ in range(4)]
                for d in range(DH):
                    idx = ib + jnp.full((L,), gbase + h * DH + d, jnp.int32)
                    parts[d % 4] = parts[d % 4] + plsc.load_gather(
                        prod, [idx])
                acc_s = (parts[0] + parts[1]) + (parts[2] + parts[3])
                sc = acc_s * 0.25
                sc = jnp.maximum(sc, ALPHA * sc)
                sc = sc * ewv
                ps[g * H + h, :] = sc
            return 0

        lax.fori_loop(0, GROUPS, lambda g, cy: score_body(g) or cy, 0)

        pltpu.make_async_copy(v_hbm.at[pl.ds(0, CHUNK)], kv_this,
                              sem_v).wait()

        @pl.when(j > 0)
        def _():
            drain_scatter()

        @pl.when(j < IDXBLK - 1)
        def _():
            pltpu.async_copy(k_hbm.at[colv.at[j + 1]], kv_other, sem_k)

        def scale_body(g):
            scores = [ps[g * H + h, :] for h in range(H)]
            m = jnp.maximum(
                jnp.maximum(jnp.maximum(scores[0], scores[1]),
                            jnp.maximum(scores[2], scores[3])),
                jnp.maximum(jnp.maximum(scores[4], scores[5]),
                            jnp.maximum(scores[6], scores[7])))
            es = [jnp.exp(sc - m) for sc in scores]
            ssum = ((es[0] + es[1]) + (es[2] + es[3])) + \
                   ((es[4] + es[5]) + (es[6] + es[7]))
            rinv = 1.0 / ssum
            probs = [e_h * rinv for e_h in es]

            dnums = lax.GatherDimensionNumbers(
                offset_dims=(), collapsed_slice_dims=(0,),
                start_index_map=(0,))
            for lane in range(L):
                e_row = g * L + lane
                bidx = jnp.full((L, 1), lane, jnp.int32)
                for h in range(H):
                    pb = lax.gather(
                        probs[h], bidx, dnums, (1,),
                        mode=lax.GatherScatterMode.PROMISE_IN_BOUNDS)
                    vrow = kv_this[e_row, pl.ds(h * DH, DH)]
                    kv_this[e_row, pl.ds(h * DH, DH)] = vrow * pb
            return 0

        lax.fori_loop(0, GROUPS, lambda g, cy: scale_body(g) or cy, 0)

        pltpu.async_copy(kv_this, acc.at[colv.at[j]], sem_s, add=True)

    def blk_body(b, carry0):
        @pl.when(b > 0)
        def _():
            drain_scatter()

        gblk = tid * NBLK + b
        pltpu.sync_copy(row_hbm.at[gblk], rowv)
        pltpu.sync_copy(col_hbm.at[gblk], colv)
        pltpu.sync_copy(ew_hbm.at[gblk], ew_v)
        pltpu.async_copy(q_hbm.at[rowv.at[0]], q_v, sem_q)
        pltpu.async_copy(k_hbm.at[colv.at[0]], kva, sem_k)

        def pair_body(j2, carry):
            process_chunk(2 * j2, kva, kvb)
            process_chunk(2 * j2 + 1, kvb, kva)
            return carry

        lax.fori_loop(0, IDXBLK // 2, pair_body, 0)
        process_chunk(IDXBLK - 1, kva, kvb)
        return carry0

    lax.fori_loop(0, NBLK, blk_body, 0)
    drain_scatter()

    plsc.subcore_barrier()
    pltpu.sync_copy(acc.at[pl.ds(s * ROWS_PER_TILE, ROWS_PER_TILE)],
                    out_hbm.at[c, pl.ds(s * ROWS_PER_TILE, ROWS_PER_TILE)])


def _sc_edge_call(q_tab, k_tab, v_tab, row3d, col3d, ew3d, zeros_n):
    mesh = plsc.VectorSubcoreMesh(core_axis_name="c", subcore_axis_name="s")
    fn = functools.partial(
        pl.kernel,
        mesh=mesh,
        compiler_params=pltpu.CompilerParams(use_tc_tiling_on_sc=False,
                                             needs_layout_passes=False),
        out_type=jax.ShapeDtypeStruct((NC, N_PAD, D), jnp.float32),
        scratch_types=[
            pltpu.VMEM((IDXBLK, CHUNK), jnp.int32),
            pltpu.VMEM((IDXBLK, CHUNK), jnp.int32),
            pltpu.VMEM((IDXBLK, CHUNK), jnp.float32),
            pltpu.VMEM((CHUNK, D), jnp.float32),
            pltpu.VMEM((CHUNK, D), jnp.float32),
            pltpu.VMEM((CHUNK, D), jnp.float32),
            pltpu.VMEM((PRODW,), jnp.float32),
            pltpu.VMEM((GROUPS * H, L), jnp.float32),
            pltpu.VMEM_SHARED((N_PAD, D), jnp.float32),
            pltpu.SemaphoreType.DMA,
            pltpu.SemaphoreType.DMA,
            pltpu.SemaphoreType.DMA,
            pltpu.SemaphoreType.DMA,
        ],
    )(_sc_edge_body)
    return fn(q_tab, k_tab, v_tab, row3d, col3d, ew3d, zeros_n)


def _out_body(p0_ref, p1_ref, x_ref, wo_ref, bo_ref, g_ref, b_ref, o_ref):
    pb = p0_ref[0] + p1_ref[0]
    y = jnp.dot(pb, wo_ref[...], preferred_element_type=jnp.float32)
    y = y + bo_ref[...] + x_ref[...]
    mu = jnp.mean(y, axis=-1, keepdims=True)
    yc = y - mu
    var = jnp.mean(yc * yc, axis=-1, keepdims=True)
    o_ref[...] = yc * lax.rsqrt(var + EPS) * g_ref[...] + b_ref[...]


def _out_call(partials, x, W_o, b_o, ln_g, ln_b):
    blk = 1000
    grid = (N // blk,)
    return pl.pallas_call(
        _out_body,
        grid=grid,
        in_specs=[
            pl.BlockSpec((1, blk, D), lambda i: (0, i, 0)),
            pl.BlockSpec((1, blk, D), lambda i: (1, i, 0)),
            pl.BlockSpec((blk, D), lambda i: (i, 0)),
            pl.BlockSpec((D, D), lambda i: (0, 0)),
            pl.BlockSpec((1, D), lambda i: (0, 0)),
            pl.BlockSpec((1, D), lambda i: (0, 0)),
            pl.BlockSpec((1, D), lambda i: (0, 0)),
        ],
        out_specs=pl.BlockSpec((blk, D), lambda i: (i, 0)),
        out_shape=jax.ShapeDtypeStruct((N, D), jnp.float32),
    )(partials, partials, x, W_o, b_o, ln_g, ln_b)


def kernel(x, edge_index, edge_attr, W_q, W_k, W_v, W_o, b_o, ln_g, ln_b):
    row3d = edge_index[0].reshape(NW * NBLK, IDXBLK, CHUNK)
    col3d = edge_index[1].reshape(NW * NBLK, IDXBLK, CHUNK)
    zeros_n = jnp.zeros((N_PAD, D), jnp.float32)
    ew3d = _ew_call(edge_attr).reshape(NW * NBLK, IDXBLK, CHUNK)
    q_tab, k_tab, v_tab = _qkv_call(x, W_q, W_k, W_v)
    partials = _sc_edge_call(q_tab, k_tab, v_tab, row3d, col3d, ew3d, zeros_n)
    return _out_call(partials, x, W_o,
                     b_o.reshape(1, D), ln_g.reshape(1, D), ln_b.reshape(1, D))
